# Initial kernel scaffold; baseline (speedup 1.0000x reference)
import jax
import jax.numpy as jnp
from jax.experimental import pallas as pl


def kernel(x, adj_matrix, W0, b0, W1, b1, W2, b2):
    raise NotImplementedError("write your pallas kernel here")


# speedup vs baseline: 1.8471x; 1.8471x over previous
import functools

import jax
import jax.numpy as jnp
from jax.experimental import pallas as pl


def _gcn_kernel(adj_ref, x_ref, w0_ref, b0_ref, w1_ref, b1_ref, w2_ref,
                b2_ref, out_ref):
    a_raw = adj_ref[0]
    n = a_raw.shape[0]
    rows = jax.lax.broadcasted_iota(jnp.int32, (n, n), 0)
    cols = jax.lax.broadcasted_iota(jnp.int32, (n, n), 1)
    on_diag = rows == cols
    a = jnp.where(jnp.logical_or(on_diag, a_raw != 0), 1.0, 0.0)
    deg = jnp.sum(a, axis=1, keepdims=True)
    dinv = jax.lax.rsqrt(deg)

    h = x_ref[0]
    layers = ((w0_ref, b0_ref, False), (w1_ref, b1_ref, False),
              (w2_ref, b2_ref, True))
    for w_ref, b_ref, is_last in layers:
        z = jnp.dot(h, w_ref[...], preferred_element_type=jnp.float32)
        z = z * dinv
        agg = jnp.dot(a, z, preferred_element_type=jnp.float32)
        h = agg * dinv + b_ref[...]
        if not is_last:
            h = jnp.where(h > 0, h, jnp.exp(jnp.minimum(h, 0.0)) - 1.0)
    out_ref[0] = h


@functools.partial(jax.jit, static_argnames=())
def kernel(x, adj_matrix, W0, b0, W1, b1, W2, b2):
    B, N, F_in = x.shape
    H = W0.shape[1]
    OUT = W2.shape[1]
    b0r = b0.reshape(1, H)
    b1r = b1.reshape(1, H)
    b2r = b2.reshape(1, OUT)

    grid = (B,)
    out = pl.pallas_call(
        _gcn_kernel,
        grid=grid,
        in_specs=[
            pl.BlockSpec((1, N, N), lambda b: (b, 0, 0)),
            pl.BlockSpec((1, N, F_in), lambda b: (b, 0, 0)),
            pl.BlockSpec((F_in, H), lambda b: (0, 0)),
            pl.BlockSpec((1, H), lambda b: (0, 0)),
            pl.BlockSpec((H, H), lambda b: (0, 0)),
            pl.BlockSpec((1, H), lambda b: (0, 0)),
            pl.BlockSpec((H, OUT), lambda b: (0, 0)),
            pl.BlockSpec((1, OUT), lambda b: (0, 0)),
        ],
        out_specs=pl.BlockSpec((1, N, OUT), lambda b: (b, 0, 0)),
        out_shape=jax.ShapeDtypeStruct((B, N, OUT), jnp.float32),
    )(adj_matrix, x, W0, b0r, W1, b1r, W2, b2r)
    return out
